# baseline (speedup 1.0000x reference)
import functools
import math

import jax
import jax.numpy as jnp
from jax import lax
from jax.experimental import pallas as pl
from jax.experimental.pallas import tpu as pltpu
from jax.experimental.pallas import tpu_sc as plsc

_VOCAB = 1000000
_DIM = 64
_B = 4096 * 200
_NW = 32
_IROW = 128
_G = 4
_CHUNK = _G * _IROW
_ROWS_PER_W = _B // _NW
_NCHUNK = _ROWS_PER_W // _CHUNK
_SCALE = math.sqrt(_DIM)

_BLKC = 16384
_NPAIR = (_VOCAB + 2 * _BLKC - 1) // (2 * _BLKC)
_LASTBLK = (_VOCAB + _BLKC - 1) // _BLKC - 1
_PROWS = _NPAIR * _BLKC
_TSUB = 512


def _scale_pack_body(a_ref, b_ref, o_ref):
    for s in range(_BLKC // _TSUB):
        sl = pl.ds(s * _TSUB, _TSUB)
        o_ref[sl, 0:_DIM] = a_ref[:, sl].T * _SCALE
        o_ref[sl, _DIM : 2 * _DIM] = b_ref[:, sl].T * _SCALE


_scale_pack = pl.pallas_call(
    _scale_pack_body,
    grid=(_NPAIR,),
    in_specs=[
        pl.BlockSpec((_DIM, _BLKC), lambda i: (0, jnp.minimum(2 * i, _LASTBLK))),
        pl.BlockSpec(
            (_DIM, _BLKC), lambda i: (0, jnp.minimum(2 * i + 1, _LASTBLK))
        ),
    ],
    out_specs=pl.BlockSpec((_BLKC, 2 * _DIM), lambda i: (i, 0)),
    out_shape=jax.ShapeDtypeStruct((_PROWS, 2 * _DIM), jnp.float32),
)

_mesh = plsc.VectorSubcoreMesh(core_axis_name="c", subcore_axis_name="s")


@functools.partial(
    pl.kernel,
    out_type=jax.ShapeDtypeStruct((_B, 2 * _DIM), jnp.float32),
    mesh=_mesh,
    compiler_params=pltpu.CompilerParams(use_tc_tiling_on_sc=False),
    scratch_types=[
        pltpu.VMEM((_CHUNK,), jnp.int32),
        pltpu.VMEM((_CHUNK,), jnp.int32),
        pltpu.VMEM((_CHUNK, _DIM), jnp.float32),
        pltpu.SemaphoreType.DMA,
    ],
)
def _emb_lookup(idx_hbm, table_hbm, out_hbm, idx_v, pidx_v, rows_v, sem):
    wid = lax.axis_index("s") * 2 + lax.axis_index("c")
    base = wid * _ROWS_PER_W

    def chunk_body(i, carry):
        row0 = base + i * _CHUNK
        pltpu.sync_copy(idx_hbm.at[pl.ds(row0, _CHUNK)], idx_v)

        def remap_body(v):
            sl = pl.ds(v * 16, 16)
            r = idx_v[sl]
            hi = (r >> 15) << 15
            mid = (r & 16383) << 1
            par = (r >> 14) & 1
            pidx_v[sl] = hi | mid | par

        plsc.parallel_loop(0, _CHUNK // 16, 1, unroll=4)(remap_body)

        copies = [
            pltpu.async_copy(
                table_hbm.at[pidx_v.at[pl.ds(j * _IROW, _IROW)]],
                rows_v.at[pl.ds(j * _IROW, _IROW)],
                sem,
            )
            for j in range(_G)
        ]
        for c in copies:
            c.wait()
        pltpu.sync_copy(
            rows_v, out_hbm.at[pl.ds(row0, _CHUNK), pl.ds(0, _DIM)]
        )
        return carry

    lax.fori_loop(0, _NCHUNK, chunk_body, 0)


def kernel(x, emb_table):
    idx = x.reshape(_B).astype(jnp.int32)
    packed = _scale_pack(emb_table.T, emb_table.T)
    table_lin = packed.reshape(2 * _PROWS, _DIM)
    out = _emb_lookup(idx, table_lin)
    return out[:, :_DIM].reshape(x.shape[0], x.shape[1], _DIM)


# speedup vs baseline: 1.0660x; 1.0660x over previous
import functools
import math

import jax
import jax.numpy as jnp
from jax import lax
from jax.experimental import pallas as pl
from jax.experimental.pallas import tpu as pltpu
from jax.experimental.pallas import tpu_sc as plsc

_VOCAB = 1000000
_DIM = 64
_B = 4096 * 200
_NW = 32
_IROW = 128
_G = 4
_CHUNK = _G * _IROW
_ROWS_PER_W = _B // _NW
_NCHUNK = _ROWS_PER_W // _CHUNK
_SCALE = math.sqrt(_DIM)

_BLKC = 16384
_NPAIR = (_VOCAB + 2 * _BLKC - 1) // (2 * _BLKC)
_LASTBLK = (_VOCAB + _BLKC - 1) // _BLKC - 1
_PROWS = _NPAIR * _BLKC
_TSUB = 512


def _scale_pack_body(a_ref, b_ref, o_ref):
    for s in range(_BLKC // _TSUB):
        sl = pl.ds(s * _TSUB, _TSUB)
        o_ref[sl, 0:_DIM] = a_ref[:, sl].T * _SCALE
        o_ref[sl, _DIM : 2 * _DIM] = b_ref[:, sl].T * _SCALE


_scale_pack = pl.pallas_call(
    _scale_pack_body,
    grid=(_NPAIR,),
    in_specs=[
        pl.BlockSpec((_DIM, _BLKC), lambda i: (0, jnp.minimum(2 * i, _LASTBLK))),
        pl.BlockSpec(
            (_DIM, _BLKC), lambda i: (0, jnp.minimum(2 * i + 1, _LASTBLK))
        ),
    ],
    out_specs=pl.BlockSpec((_BLKC, 2 * _DIM), lambda i: (i, 0)),
    out_shape=jax.ShapeDtypeStruct((_PROWS, 2 * _DIM), jnp.float32),
)

_mesh = plsc.VectorSubcoreMesh(core_axis_name="c", subcore_axis_name="s")


@functools.partial(
    pl.kernel,
    out_type=jax.ShapeDtypeStruct((_B, 2 * _DIM), jnp.float32),
    mesh=_mesh,
    compiler_params=pltpu.CompilerParams(use_tc_tiling_on_sc=False),
    scratch_types=[
        pltpu.VMEM((2, _CHUNK), jnp.int32),
        pltpu.VMEM((2, _CHUNK), jnp.int32),
        pltpu.VMEM((2, _CHUNK, _DIM), jnp.float32),
        pltpu.SemaphoreType.DMA,
        pltpu.SemaphoreType.DMA,
    ],
)
def _emb_lookup(idx_hbm, table_hbm, out_hbm, idx_v, pidx_v, rows_v, gsem, osem):
    wid = lax.axis_index("s") * 2 + lax.axis_index("c")
    base = wid * _ROWS_PER_W

    def fire(i, buf):
        row0 = base + i * _CHUNK
        pltpu.sync_copy(idx_hbm.at[pl.ds(row0, _CHUNK)], idx_v.at[buf])

        def remap_body(v):
            sl = pl.ds(v * 16, 16)
            r = idx_v[buf, sl]
            hi = (r >> 15) << 15
            mid = (r & 16383) << 1
            par = (r >> 14) & 1
            pidx_v[buf, sl] = hi | mid | par

        plsc.parallel_loop(0, _CHUNK // 16, 1, unroll=4)(remap_body)
        for j in range(_G):
            pltpu.async_copy(
                table_hbm.at[pidx_v.at[buf, pl.ds(j * _IROW, _IROW)]],
                rows_v.at[buf, pl.ds(j * _IROW, _IROW)],
                gsem,
            )

    def wait_gathers(buf):
        for j in range(_G):
            pltpu.make_async_copy(
                table_hbm.at[pidx_v.at[buf, pl.ds(j * _IROW, _IROW)]],
                rows_v.at[buf, pl.ds(j * _IROW, _IROW)],
                gsem,
            ).wait()

    def drain_out(buf):
        pltpu.make_async_copy(
            rows_v.at[buf],
            out_hbm.at[pl.ds(base, _CHUNK), pl.ds(0, _DIM)],
            osem,
        ).wait()

    fire(0, 0)

    def chunk_body(i, carry):
        buf = lax.rem(i, 2)
        nbuf = 1 - buf
        row0 = base + i * _CHUNK
        wait_gathers(buf)
        pltpu.async_copy(
            rows_v.at[buf],
            out_hbm.at[pl.ds(row0, _CHUNK), pl.ds(0, _DIM)],
            osem,
        )

        @pl.when(i >= 1)
        def _():
            drain_out(nbuf)

        @pl.when(i + 1 < _NCHUNK)
        def _():
            fire(i + 1, nbuf)

        return carry

    lax.fori_loop(0, _NCHUNK, chunk_body, 0)
    drain_out(lax.rem(_NCHUNK - 1, 2))


def kernel(x, emb_table):
    idx = x.reshape(_B).astype(jnp.int32)
    packed = _scale_pack(emb_table.T, emb_table.T)
    table_lin = packed.reshape(2 * _PROWS, _DIM)
    out = _emb_lookup(idx, table_lin)
    return out[:, :_DIM].reshape(x.shape[0], x.shape[1], _DIM)


# speedup vs baseline: 1.0804x; 1.0135x over previous
import functools
import math

import jax
import jax.numpy as jnp
from jax import lax
from jax.experimental import pallas as pl
from jax.experimental.pallas import tpu as pltpu
from jax.experimental.pallas import tpu_sc as plsc

_VOCAB = 1000000
_DIM = 64
_B = 4096 * 200
_NW = 32
_IROW = 128
_G = 5
_CHUNK = _G * _IROW
_ROWS_PER_W = _B // _NW
_NCHUNK = _ROWS_PER_W // _CHUNK
_SCALE = math.sqrt(_DIM)

_BLKC = 16384
_NPAIR = (_VOCAB + 2 * _BLKC - 1) // (2 * _BLKC)
_LASTBLK = (_VOCAB + _BLKC - 1) // _BLKC - 1
_PROWS = _NPAIR * _BLKC
_TSUB = 512


def _scale_pack_body(a_ref, b_ref, o_ref):
    for s in range(_BLKC // _TSUB):
        sl = pl.ds(s * _TSUB, _TSUB)
        o_ref[sl, 0:_DIM] = a_ref[:, sl].T * _SCALE
        o_ref[sl, _DIM : 2 * _DIM] = b_ref[:, sl].T * _SCALE


_scale_pack = pl.pallas_call(
    _scale_pack_body,
    grid=(_NPAIR,),
    in_specs=[
        pl.BlockSpec((_DIM, _BLKC), lambda i: (0, jnp.minimum(2 * i, _LASTBLK))),
        pl.BlockSpec(
            (_DIM, _BLKC), lambda i: (0, jnp.minimum(2 * i + 1, _LASTBLK))
        ),
    ],
    out_specs=pl.BlockSpec((_BLKC, 2 * _DIM), lambda i: (i, 0)),
    out_shape=jax.ShapeDtypeStruct((_PROWS, 2 * _DIM), jnp.float32),
)

_mesh = plsc.VectorSubcoreMesh(core_axis_name="c", subcore_axis_name="s")


@functools.partial(
    pl.kernel,
    out_type=jax.ShapeDtypeStruct((_B, 2 * _DIM), jnp.float32),
    mesh=_mesh,
    compiler_params=pltpu.CompilerParams(use_tc_tiling_on_sc=False),
    scratch_types=[
        pltpu.VMEM((2, _CHUNK), jnp.int32),
        pltpu.VMEM((2, _CHUNK), jnp.int32),
        pltpu.VMEM((2, _CHUNK, _DIM), jnp.float32),
        pltpu.SemaphoreType.DMA,
        pltpu.SemaphoreType.DMA,
    ],
)
def _emb_lookup(idx_hbm, table_hbm, out_hbm, idx_v, pidx_v, rows_v, gsem, osem):
    wid = lax.axis_index("s") * 2 + lax.axis_index("c")
    base = wid * _ROWS_PER_W

    def fire(i, buf):
        row0 = base + i * _CHUNK
        pltpu.sync_copy(idx_hbm.at[pl.ds(row0, _CHUNK)], idx_v.at[buf])

        def remap_body(v):
            sl = pl.ds(v * 16, 16)
            r = idx_v[buf, sl]
            hi = (r >> 15) << 15
            mid = (r & 16383) << 1
            par = (r >> 14) & 1
            pidx_v[buf, sl] = hi | mid | par

        plsc.parallel_loop(0, _CHUNK // 16, 1, unroll=4)(remap_body)
        for j in range(_G):
            pltpu.async_copy(
                table_hbm.at[pidx_v.at[buf, pl.ds(j * _IROW, _IROW)]],
                rows_v.at[buf, pl.ds(j * _IROW, _IROW)],
                gsem,
            )

    def wait_gathers(buf):
        for j in range(_G):
            pltpu.make_async_copy(
                table_hbm.at[pidx_v.at[buf, pl.ds(j * _IROW, _IROW)]],
                rows_v.at[buf, pl.ds(j * _IROW, _IROW)],
                gsem,
            ).wait()

    def drain_out(buf):
        pltpu.make_async_copy(
            rows_v.at[buf],
            out_hbm.at[pl.ds(base, _CHUNK), pl.ds(0, _DIM)],
            osem,
        ).wait()

    fire(0, 0)

    def chunk_body(i, carry):
        buf = lax.rem(i, 2)
        nbuf = 1 - buf
        row0 = base + i * _CHUNK
        wait_gathers(buf)
        pltpu.async_copy(
            rows_v.at[buf],
            out_hbm.at[pl.ds(row0, _CHUNK), pl.ds(0, _DIM)],
            osem,
        )

        @pl.when(i >= 1)
        def _():
            drain_out(nbuf)

        @pl.when(i + 1 < _NCHUNK)
        def _():
            fire(i + 1, nbuf)

        return carry

    lax.fori_loop(0, _NCHUNK, chunk_body, 0)
    drain_out(lax.rem(_NCHUNK - 1, 2))


def kernel(x, emb_table):
    idx = x.reshape(_B).astype(jnp.int32)
    packed = _scale_pack(emb_table.T, emb_table.T)
    table_lin = packed.reshape(2 * _PROWS, _DIM)
    out = _emb_lookup(idx, table_lin)
    return out[:, :_DIM].reshape(x.shape[0], x.shape[1], _DIM)
